# baseline (speedup 1.0000x reference)
import functools

import jax
import jax.numpy as jnp
from jax import lax
from jax.experimental import pallas as pl
from jax.experimental.pallas import tpu as pltpu
from jax.experimental.pallas import tpu_sc as plsc

NC = 2
NS = 16
NW = NC * NS
LANES = 16
CHUNK = 128
GROUP = 40


def _round_up(n, m):
    return ((n + m - 1) // m) * m


@functools.lru_cache(maxsize=None)
def _make_deg_kernel(n_pad, kd):
    mesh = plsc.VectorSubcoreMesh(core_axis_name="c", subcore_axis_name="s")
    rows_per_tile = n_pad // NS

    @functools.partial(
        pl.kernel,
        out_type=jax.ShapeDtypeStruct((2, n_pad), jnp.float32),
        mesh=mesh,
        scratch_types=[
            pltpu.VMEM((kd, CHUNK), jnp.int32),
            pltpu.VMEM((CHUNK,), jnp.float32),
            pltpu.VMEM((rows_per_tile,), jnp.float32),
            pltpu.VMEM_SHARED((n_pad,), jnp.float32),
        ],
    )
    def deg_kernel(src_hbm, dst_hbm, out_hbm, idx_v, ones_v, z_v, cnt_sh):
        c = lax.axis_index("c")
        s = lax.axis_index("s")

        @pl.when(c == 0)
        def _():
            pltpu.sync_copy(src_hbm.at[s], idx_v)

        @pl.when(c != 0)
        def _():
            pltpu.sync_copy(dst_hbm.at[s], idx_v)

        zeros16 = jnp.zeros((LANES,), jnp.float32)
        ones16 = jnp.ones((LANES,), jnp.float32)
        for j in range(CHUNK // LANES):
            ones_v[pl.ds(j * LANES, LANES)] = ones16

        def zbody(i, carry):
            z_v[pl.ds(i * LANES, LANES)] = zeros16
            return carry

        lax.fori_loop(0, rows_per_tile // LANES, zbody, 0)
        pltpu.sync_copy(z_v, cnt_sh.at[pl.ds(s * rows_per_tile, rows_per_tile)])
        plsc.subcore_barrier()

        def body(k, carry):
            pltpu.sync_copy(ones_v, cnt_sh.at[idx_v.at[k]], add=True)
            return carry

        lax.fori_loop(0, kd, body, 0)
        plsc.subcore_barrier()
        pltpu.sync_copy(
            cnt_sh.at[pl.ds(s * rows_per_tile, rows_per_tile)],
            out_hbm.at[c, pl.ds(s * rows_per_tile, rows_per_tile)],
        )

    return deg_kernel


@functools.lru_cache(maxsize=None)
def _make_agg_kernel(n_pad, k0, k1, d):
    mesh = plsc.VectorSubcoreMesh(core_axis_name="c", subcore_axis_name="s")
    rows_per_tile = n_pad // NS

    @functools.partial(
        pl.kernel,
        out_type=jax.ShapeDtypeStruct((NC, n_pad, d), jnp.float32),
        mesh=mesh,
        scratch_types=[
            pltpu.VMEM((2, max(k0, k1), CHUNK), jnp.int32),
            pltpu.VMEM((CHUNK, d), jnp.float32),
            pltpu.VMEM_SHARED((n_pad, d), jnp.float32),
            pltpu.SemaphoreType.DMA,
        ],
    )
    def agg_kernel(ya_hbm, yb_hbm, edges_hbm, out_hbm, idx_v, rows0, acc_sh,
                   sem0):
        c = lax.axis_index("c")
        s = lax.axis_index("s")
        wid = c * NS + s

        zeros16 = jnp.zeros((LANES,), jnp.float32)

        def zbody(r, carry):
            for j in range(d // LANES):
                rows0[r, pl.ds(j * LANES, LANES)] = zeros16
            return carry

        lax.fori_loop(0, CHUNK, zbody, 0)
        base = s * rows_per_tile
        off = 0
        while off < rows_per_tile:
            sz = min(CHUNK, rows_per_tile - off)
            pltpu.sync_copy(rows0.at[pl.ds(0, sz)],
                            acc_sh.at[pl.ds(base + off, sz)])
            off += sz
        plsc.subcore_barrier()

        pltpu.sync_copy(edges_hbm.at[wid], idx_v)

        def run(y_hbm, lo, hi):
            def body(k, carry2):
                pltpu.async_copy(
                    y_hbm.at[idx_v.at[0, k]], rows0, sem0).wait()
                pltpu.sync_copy(rows0, acc_sh.at[idx_v.at[1, k]], add=True)
                return carry2

            lax.fori_loop(lo, hi, body, 0)

        @pl.when(c == 0)
        def _():
            run(ya_hbm, 0, k0)

        @pl.when(c == 1)
        def _():
            run(yb_hbm, 0, k1)

        plsc.subcore_barrier()
        pltpu.sync_copy(
            acc_sh.at[pl.ds(base, rows_per_tile)],
            out_hbm.at[c, pl.ds(base, rows_per_tile)],
        )

    return agg_kernel


def _tc1_body(x_ref, dego_ref, w_ref, o_ref, o2_ref):
    nsrc = lax.rsqrt(jnp.maximum(dego_ref[...], 1.0))
    y = jnp.dot(x_ref[...] * nsrc, w_ref[...],
                preferred_element_type=jnp.float32)
    o_ref[...] = y
    o2_ref[...] = y


def _tc2_body(p_ref, degi_ref, dego_ref, b1_ref, w_ref, o_ref, o2_ref):
    ndst = lax.rsqrt(jnp.maximum(degi_ref[...], 1.0))
    nsrc = lax.rsqrt(jnp.maximum(dego_ref[...], 1.0))
    h = jnp.maximum((p_ref[0] + p_ref[1]) * ndst + b1_ref[...], 0.0)
    y = jnp.dot(h * nsrc, w_ref[...], preferred_element_type=jnp.float32)
    o_ref[...] = y
    o2_ref[...] = y


def _tc3_body(p_ref, degi_ref, b2_ref, x_ref, o_ref):
    ndst = lax.rsqrt(jnp.maximum(degi_ref[...], 1.0))
    o_ref[...] = (p_ref[0] + p_ref[1]) * ndst + b2_ref[...] + x_ref[...]


def _tc1(x_pad, deg_o, w, blk):
    n_pad, d = x_pad.shape
    grid = n_pad // blk
    return pl.pallas_call(
        _tc1_body,
        grid=(grid,),
        in_specs=[
            pl.BlockSpec((blk, d), lambda i: (i, 0)),
            pl.BlockSpec((blk, 1), lambda i: (i, 0)),
            pl.BlockSpec((d, d), lambda i: (0, 0)),
        ],
        out_specs=[pl.BlockSpec((blk, d), lambda i: (i, 0)),
                   pl.BlockSpec((blk, d), lambda i: (i, 0))],
        out_shape=[jax.ShapeDtypeStruct((n_pad, d), jnp.float32),
                   jax.ShapeDtypeStruct((n_pad, d), jnp.float32)],
    )(x_pad, deg_o, w)


def _tc2(p, deg_i, deg_o, b1, w, blk):
    _, n_pad, d = p.shape
    grid = n_pad // blk
    return pl.pallas_call(
        _tc2_body,
        grid=(grid,),
        in_specs=[
            pl.BlockSpec((NC, blk, d), lambda i: (0, i, 0)),
            pl.BlockSpec((blk, 1), lambda i: (i, 0)),
            pl.BlockSpec((blk, 1), lambda i: (i, 0)),
            pl.BlockSpec((1, d), lambda i: (0, 0)),
            pl.BlockSpec((d, d), lambda i: (0, 0)),
        ],
        out_specs=[pl.BlockSpec((blk, d), lambda i: (i, 0)),
                   pl.BlockSpec((blk, d), lambda i: (i, 0))],
        out_shape=[jax.ShapeDtypeStruct((n_pad, d), jnp.float32),
                   jax.ShapeDtypeStruct((n_pad, d), jnp.float32)],
    )(p, deg_i, deg_o, b1, w)


def _tc3(p, deg_i, b2, x_pad, blk):
    _, n_pad, d = p.shape
    grid = n_pad // blk
    return pl.pallas_call(
        _tc3_body,
        grid=(grid,),
        in_specs=[
            pl.BlockSpec((NC, blk, d), lambda i: (0, i, 0)),
            pl.BlockSpec((blk, 1), lambda i: (i, 0)),
            pl.BlockSpec((1, d), lambda i: (0, 0)),
            pl.BlockSpec((blk, d), lambda i: (i, 0)),
        ],
        out_specs=pl.BlockSpec((blk, d), lambda i: (i, 0)),
        out_shape=jax.ShapeDtypeStruct((n_pad, d), jnp.float32),
    )(p, deg_i, b2, x_pad)


def kernel(x, edge_index, W1, b1, W2, b2):
    n, d = x.shape
    e = edge_index.shape[1]

    n_pad = _round_up(n + 1, NS * LANES)
    dummy = n_pad - 1

    n_chunks = ((e + CHUNK - 1) // CHUNK + 1) // 2 * 2
    k0 = int(n_chunks * 0.65) // NS
    k1 = (n_chunks - NS * k0 + NS - 1) // NS
    kd = (n_chunks + NS - 1) // NS
    e_pad = NS * (k0 + k1) * CHUNK

    pad = jnp.full((e_pad - e,), dummy, jnp.int32)
    src = jnp.concatenate([edge_index[0], pad])
    dst = jnp.concatenate([edge_index[1], pad])

    kmax = max(k0, k1)

    def _pack(a):
        c0 = a[:NS * k0 * CHUNK].reshape(NS, k0, CHUNK)
        c1 = a[NS * k0 * CHUNK:].reshape(NS, k1, CHUNK)
        parts = []
        for ci, ki in ((c0, k0), (c1, k1)):
            if ki < kmax:
                ci = jnp.concatenate(
                    [ci, jnp.full((NS, kmax - ki, CHUNK), dummy, jnp.int32)],
                    axis=1)
            parts.append(ci)
        return jnp.concatenate(parts, axis=0)

    edges_packed = jnp.stack([_pack(src), _pack(dst)], axis=1)

    e_deg = NS * kd * CHUNK
    pad_d = jnp.full((e_deg - e,), dummy, jnp.int32)
    src_d = jnp.concatenate([edge_index[0], pad_d]).reshape(NS, kd, CHUNK)
    dst_d = jnp.concatenate([edge_index[1], pad_d]).reshape(NS, kd, CHUNK)
    x_pad = jnp.concatenate(
        [x, jnp.zeros((n_pad - n, d), x.dtype)], axis=0)

    deg = _make_deg_kernel(n_pad, kd)(src_d, dst_d)
    deg_o = deg[0][:, None]
    deg_i = deg[1][:, None]

    blk = n_pad // 4
    agg = _make_agg_kernel(n_pad, k0, k1, d)

    y1a, y1b = _tc1(x_pad, deg_o, W1, blk)
    p1 = agg(y1a, y1b, edges_packed)
    y2a, y2b = _tc2(p1, deg_i, deg_o, b1[None, :], W2, blk)
    p2 = agg(y2a, y2b, edges_packed)
    out = _tc3(p2, deg_i, b2[None, :], x_pad, blk)
    return out[:n]


# speedup vs baseline: 1.0498x; 1.0498x over previous
import functools

import jax
import jax.numpy as jnp
from jax import lax
from jax.experimental import pallas as pl
from jax.experimental.pallas import tpu as pltpu
from jax.experimental.pallas import tpu_sc as plsc

NC = 2
NS = 16
NW = NC * NS
LANES = 16
CHUNK = 128
GROUP = 40


def _round_up(n, m):
    return ((n + m - 1) // m) * m


@functools.lru_cache(maxsize=None)
def _make_deg_kernel(n_pad, kd):
    mesh = plsc.VectorSubcoreMesh(core_axis_name="c", subcore_axis_name="s")
    rows_per_tile = n_pad // NS

    @functools.partial(
        pl.kernel,
        out_type=jax.ShapeDtypeStruct((2, n_pad), jnp.float32),
        mesh=mesh,
        scratch_types=[
            pltpu.VMEM((kd, CHUNK), jnp.int32),
            pltpu.VMEM((CHUNK,), jnp.float32),
            pltpu.VMEM((rows_per_tile,), jnp.float32),
            pltpu.VMEM_SHARED((n_pad,), jnp.float32),
        ],
    )
    def deg_kernel(src_hbm, dst_hbm, out_hbm, idx_v, ones_v, z_v, cnt_sh):
        c = lax.axis_index("c")
        s = lax.axis_index("s")

        @pl.when(c == 0)
        def _():
            pltpu.sync_copy(src_hbm.at[s], idx_v)

        @pl.when(c != 0)
        def _():
            pltpu.sync_copy(dst_hbm.at[s], idx_v)

        zeros16 = jnp.zeros((LANES,), jnp.float32)
        ones16 = jnp.ones((LANES,), jnp.float32)
        for j in range(CHUNK // LANES):
            ones_v[pl.ds(j * LANES, LANES)] = ones16

        def zbody(i, carry):
            z_v[pl.ds(i * LANES, LANES)] = zeros16
            return carry

        lax.fori_loop(0, rows_per_tile // LANES, zbody, 0)
        pltpu.sync_copy(z_v, cnt_sh.at[pl.ds(s * rows_per_tile, rows_per_tile)])
        plsc.subcore_barrier()

        def body(k, carry):
            pltpu.sync_copy(ones_v, cnt_sh.at[idx_v.at[k]], add=True)
            return carry

        lax.fori_loop(0, kd, body, 0)
        plsc.subcore_barrier()
        pltpu.sync_copy(
            cnt_sh.at[pl.ds(s * rows_per_tile, rows_per_tile)],
            out_hbm.at[c, pl.ds(s * rows_per_tile, rows_per_tile)],
        )

    return deg_kernel


@functools.lru_cache(maxsize=None)
def _make_agg_kernel(n_pad, k0, k1, d):
    mesh = plsc.VectorSubcoreMesh(core_axis_name="c", subcore_axis_name="s")
    rows_per_tile = n_pad // NS

    @functools.partial(
        pl.kernel,
        out_type=jax.ShapeDtypeStruct((NC, n_pad, d), jnp.float32),
        mesh=mesh,
        scratch_types=[
            pltpu.VMEM((2, max(k0, k1), CHUNK), jnp.int32),
            pltpu.VMEM((CHUNK, d), jnp.float32),
            pltpu.VMEM_SHARED((n_pad, d), jnp.float32),
            pltpu.SemaphoreType.DMA,
        ],
    )
    def agg_kernel(ya_hbm, yb_hbm, edges_hbm, out_hbm, idx_v, rows0, acc_sh,
                   sem0):
        c = lax.axis_index("c")
        s = lax.axis_index("s")
        wid = c * NS + s

        zeros16 = jnp.zeros((LANES,), jnp.float32)

        def zbody(r, carry):
            for j in range(d // LANES):
                rows0[r, pl.ds(j * LANES, LANES)] = zeros16
            return carry

        lax.fori_loop(0, CHUNK, zbody, 0)
        base = s * rows_per_tile
        off = 0
        while off < rows_per_tile:
            sz = min(CHUNK, rows_per_tile - off)
            pltpu.sync_copy(rows0.at[pl.ds(0, sz)],
                            acc_sh.at[pl.ds(base + off, sz)])
            off += sz
        plsc.subcore_barrier()

        pltpu.sync_copy(edges_hbm.at[wid], idx_v)

        def run(y_hbm, lo, hi):
            def body(k, carry2):
                pltpu.async_copy(
                    y_hbm.at[idx_v.at[0, k]], rows0, sem0).wait()
                pltpu.sync_copy(rows0, acc_sh.at[idx_v.at[1, k]], add=True)
                return carry2

            lax.fori_loop(lo, hi, body, 0)

        @pl.when(c == 0)
        def _():
            run(ya_hbm, 0, k0)

        @pl.when(c == 1)
        def _():
            run(yb_hbm, 0, k1)

        plsc.subcore_barrier()
        pltpu.sync_copy(
            acc_sh.at[pl.ds(base, rows_per_tile)],
            out_hbm.at[c, pl.ds(base, rows_per_tile)],
        )

    return agg_kernel


def _tc1_body(x_ref, dego_ref, w_ref, o_ref, o2_ref):
    nsrc = lax.rsqrt(jnp.maximum(dego_ref[...], 1.0))
    y = jnp.dot(x_ref[...] * nsrc, w_ref[...],
                preferred_element_type=jnp.float32)
    o_ref[...] = y
    o2_ref[...] = y


def _tc2_body(p_ref, degi_ref, dego_ref, b1_ref, w_ref, o_ref, o2_ref):
    ndst = lax.rsqrt(jnp.maximum(degi_ref[...], 1.0))
    nsrc = lax.rsqrt(jnp.maximum(dego_ref[...], 1.0))
    h = jnp.maximum((p_ref[0] + p_ref[1]) * ndst + b1_ref[...], 0.0)
    y = jnp.dot(h * nsrc, w_ref[...], preferred_element_type=jnp.float32)
    o_ref[...] = y
    o2_ref[...] = y


def _tc3_body(p_ref, degi_ref, b2_ref, x_ref, o_ref):
    ndst = lax.rsqrt(jnp.maximum(degi_ref[...], 1.0))
    o_ref[...] = (p_ref[0] + p_ref[1]) * ndst + b2_ref[...] + x_ref[...]


def _tc1(x_pad, deg_o, w, blk):
    n_pad, d = x_pad.shape
    grid = n_pad // blk
    return pl.pallas_call(
        _tc1_body,
        grid=(grid,),
        in_specs=[
            pl.BlockSpec((blk, d), lambda i: (i, 0)),
            pl.BlockSpec((blk, 1), lambda i: (i, 0)),
            pl.BlockSpec((d, d), lambda i: (0, 0)),
        ],
        out_specs=[pl.BlockSpec((blk, d), lambda i: (i, 0)),
                   pl.BlockSpec((blk, d), lambda i: (i, 0))],
        out_shape=[jax.ShapeDtypeStruct((n_pad, d), jnp.float32),
                   jax.ShapeDtypeStruct((n_pad, d), jnp.float32)],
    )(x_pad, deg_o, w)


def _tc2(p, deg_i, deg_o, b1, w, blk):
    _, n_pad, d = p.shape
    grid = n_pad // blk
    return pl.pallas_call(
        _tc2_body,
        grid=(grid,),
        in_specs=[
            pl.BlockSpec((NC, blk, d), lambda i: (0, i, 0)),
            pl.BlockSpec((blk, 1), lambda i: (i, 0)),
            pl.BlockSpec((blk, 1), lambda i: (i, 0)),
            pl.BlockSpec((1, d), lambda i: (0, 0)),
            pl.BlockSpec((d, d), lambda i: (0, 0)),
        ],
        out_specs=[pl.BlockSpec((blk, d), lambda i: (i, 0)),
                   pl.BlockSpec((blk, d), lambda i: (i, 0))],
        out_shape=[jax.ShapeDtypeStruct((n_pad, d), jnp.float32),
                   jax.ShapeDtypeStruct((n_pad, d), jnp.float32)],
    )(p, deg_i, deg_o, b1, w)


def _tc3(p, deg_i, b2, x_pad, blk):
    _, n_pad, d = p.shape
    grid = n_pad // blk
    return pl.pallas_call(
        _tc3_body,
        grid=(grid,),
        in_specs=[
            pl.BlockSpec((NC, blk, d), lambda i: (0, i, 0)),
            pl.BlockSpec((blk, 1), lambda i: (i, 0)),
            pl.BlockSpec((1, d), lambda i: (0, 0)),
            pl.BlockSpec((blk, d), lambda i: (i, 0)),
        ],
        out_specs=pl.BlockSpec((blk, d), lambda i: (i, 0)),
        out_shape=jax.ShapeDtypeStruct((n_pad, d), jnp.float32),
    )(p, deg_i, b2, x_pad)


def kernel(x, edge_index, W1, b1, W2, b2):
    n, d = x.shape
    e = edge_index.shape[1]

    n_pad = _round_up(n + 1, NS * LANES)
    dummy = n_pad - 1

    n_chunks = ((e + CHUNK - 1) // CHUNK + 1) // 2 * 2
    k0 = int(n_chunks * 0.61) // NS
    k1 = (n_chunks - NS * k0 + NS - 1) // NS
    kd = (n_chunks + NS - 1) // NS
    e_pad = NS * (k0 + k1) * CHUNK

    pad = jnp.full((e_pad - e,), dummy, jnp.int32)
    src = jnp.concatenate([edge_index[0], pad])
    dst = jnp.concatenate([edge_index[1], pad])

    kmax = max(k0, k1)

    def _pack(a):
        c0 = a[:NS * k0 * CHUNK].reshape(NS, k0, CHUNK)
        c1 = a[NS * k0 * CHUNK:].reshape(NS, k1, CHUNK)
        parts = []
        for ci, ki in ((c0, k0), (c1, k1)):
            if ki < kmax:
                ci = jnp.concatenate(
                    [ci, jnp.full((NS, kmax - ki, CHUNK), dummy, jnp.int32)],
                    axis=1)
            parts.append(ci)
        return jnp.concatenate(parts, axis=0)

    edges_packed = jnp.stack([_pack(src), _pack(dst)], axis=1)

    e_deg = NS * kd * CHUNK
    pad_d = jnp.full((e_deg - e,), dummy, jnp.int32)
    src_d = jnp.concatenate([edge_index[0], pad_d]).reshape(NS, kd, CHUNK)
    dst_d = jnp.concatenate([edge_index[1], pad_d]).reshape(NS, kd, CHUNK)
    x_pad = jnp.concatenate(
        [x, jnp.zeros((n_pad - n, d), x.dtype)], axis=0)

    deg = _make_deg_kernel(n_pad, kd)(src_d, dst_d)
    deg_o = deg[0][:, None]
    deg_i = deg[1][:, None]

    blk = n_pad // 4
    agg = _make_agg_kernel(n_pad, k0, k1, d)

    y1a, y1b = _tc1(x_pad, deg_o, W1, blk)
    p1 = agg(y1a, y1b, edges_packed)
    y2a, y2b = _tc2(p1, deg_i, deg_o, b1[None, :], W2, blk)
    p2 = agg(y2a, y2b, edges_packed)
    out = _tc3(p2, deg_i, b2[None, :], x_pad, blk)
    return out[:n]
